# Initial kernel scaffold; baseline (speedup 1.0000x reference)
import jax
import jax.numpy as jnp
from jax.experimental import pallas as pl


def kernel(h, pos, vel, g, params):
    raise NotImplementedError("write your pallas kernel here")


# speedup vs baseline: 1.0733x; 1.0733x over previous
import functools

import jax
import jax.numpy as jnp
from jax.experimental import pallas as pl
from jax.experimental.pallas import tpu as pltpu

DT = 0.01
DH = 0.1
R2 = 1.5 * 1.5
COORDS_WEIGHT = 1.0
NP = 32


def _silu(x):
    return x * jax.nn.sigmoid(x)


def _enflow_kernel(n_layers, n_atoms, bb,
                   h_ref, pos_ref, vel_ref, g_ref,
                   wa_ref, wb_ref, wr_ref, be1_ref,
                   we2_ref, be2_ref, wc1_ref, bc1_ref, wc2_ref,
                   wn1h_ref, wn1a_ref, bn1_ref, wn2_ref, bn2_ref,
                   ws_ref, bs_ref,
                   h_out, pos_out, vel_out, g_out, s_out):
    nf = h_ref.shape[-1]
    h = h_ref[...]
    pos = pos_ref[...]
    vel = vel_ref[...]
    g = g_ref[...]

    ii = jax.lax.broadcasted_iota(jnp.int32, (NP, NP), 0)
    jj = jax.lax.broadcasted_iota(jnp.int32, (NP, NP), 1)
    base_ok = (ii != jj) & (jj < n_atoms)

    s_acc = jnp.zeros((bb * NP, 1), jnp.float32)

    for l in range(n_layers):
        diff = pos[:, :, None, :] - pos[:, None, :, :]
        radial = jnp.sum(diff * diff, axis=-1)
        mask = (radial < R2) & base_ok[None]
        maskf = mask.astype(jnp.float32)[..., None]

        hf2 = h.reshape(bb * NP, nf)
        a = (jnp.dot(hf2, wa_ref[l], preferred_element_type=jnp.float32)
             + be1_ref[l]).reshape(bb, NP, 1, nf)
        b = jnp.dot(hf2, wb_ref[l],
                    preferred_element_type=jnp.float32).reshape(bb, 1, NP, nf)
        pre = a + b + radial[..., None] * wr_ref[l]
        m = _silu(pre).reshape(bb * NP * NP, nf)
        m = _silu(jnp.dot(m, we2_ref[l],
                          preferred_element_type=jnp.float32) + be2_ref[l])
        m4 = m.reshape(bb, NP, NP, nf) * maskf
        agg = jnp.sum(m4, axis=2)
        mflat = m4.reshape(bb * NP * NP, nf)
        c1 = _silu(jnp.dot(mflat, wc1_ref[l],
                           preferred_element_type=jnp.float32) + bc1_ref[l])
        cm = jnp.dot(c1, wc2_ref[l],
                     preferred_element_type=jnp.float32)
        cm4 = cm.reshape(bb, NP, NP, 1) * maskf
        force = jnp.sum(diff * cm4, axis=2) * COORDS_WEIGHT

        aggf = agg.reshape(bb * NP, nf)
        hn = _silu(jnp.dot(hf2, wn1h_ref[l], preferred_element_type=jnp.float32)
                   + jnp.dot(aggf, wn1a_ref[l], preferred_element_type=jnp.float32)
                   + bn1_ref[l])
        hforce = (jnp.dot(hn, wn2_ref[l], preferred_element_type=jnp.float32)
                  + bn2_ref[l])
        s = (jnp.dot(aggf, ws_ref[l], preferred_element_type=jnp.float32)
             + bs_ref[l])

        s3 = s.reshape(bb, NP, 1)
        vel = jnp.exp(s3) * vel + force * DT
        pos = pos + vel * DT
        g = g + hforce.reshape(bb, NP, nf) * DH
        h = h + g * DH
        s_acc = s_acc + s

    h_out[...] = h
    pos_out[...] = pos
    vel_out[...] = vel
    g_out[...] = g
    s_out[...] = s_acc.reshape(bb, NP, 1)


def kernel(h, pos, vel, g, params):
    B, N, nf = h.shape
    n_layers = len(params)
    bb = 4
    pad = NP - N

    hp = jnp.pad(h, ((0, 0), (0, pad), (0, 0)))
    posp = jnp.pad(pos, ((0, 0), (0, pad), (0, 0)))
    velp = jnp.pad(vel, ((0, 0), (0, pad), (0, 0)))
    gp = jnp.pad(g, ((0, 0), (0, pad), (0, 0)))

    st = lambda name: jnp.stack([p[name] for p in params])
    we1 = st("We1")
    wa = we1[:, :nf]
    wb = we1[:, nf:2 * nf]
    wr = we1[:, 2 * nf:]
    be1 = st("be1")[:, None, :]
    we2 = st("We2")
    be2 = st("be2")[:, None, :]
    wc1 = st("Wc1")
    bc1 = st("bc1")[:, None, :]
    wc2 = st("Wc2")
    wn1 = st("Wn1")
    wn1h = wn1[:, :nf]
    wn1a = wn1[:, nf:]
    bn1 = st("bn1")[:, None, :]
    wn2 = st("Wn2")
    bn2 = st("bn2")[:, None, :]
    ws = st("Ws")
    bs = st("bs")[:, :, None]

    def wspec(x):
        return pl.BlockSpec(x.shape, lambda i: (0,) * x.ndim)

    def bspec(last):
        return pl.BlockSpec((bb, NP, last), lambda i: (i, 0, 0))

    weights = (wa, wb, wr, be1, we2, be2, wc1, bc1, wc2,
               wn1h, wn1a, bn1, wn2, bn2, ws, bs)

    outs = pl.pallas_call(
        functools.partial(_enflow_kernel, n_layers, N, bb),
        grid=(B // bb,),
        in_specs=[bspec(nf), bspec(3), bspec(3), bspec(nf)]
                 + [wspec(w) for w in weights],
        out_specs=[bspec(nf), bspec(3), bspec(3), bspec(nf), bspec(1)],
        out_shape=[
            jax.ShapeDtypeStruct((B, NP, nf), jnp.float32),
            jax.ShapeDtypeStruct((B, NP, 3), jnp.float32),
            jax.ShapeDtypeStruct((B, NP, 3), jnp.float32),
            jax.ShapeDtypeStruct((B, NP, nf), jnp.float32),
            jax.ShapeDtypeStruct((B, NP, 1), jnp.float32),
        ],
        compiler_params=pltpu.CompilerParams(
            dimension_semantics=("parallel",)),
    )(hp, posp, velp, gp, *weights)

    h_o, pos_o, vel_o, g_o, s_o = outs
    ldj = jnp.sum(s_o[:, :N])
    return (h_o[:, :N], pos_o[:, :N], vel_o[:, :N], g_o[:, :N], ldj)
